# Initial kernel scaffold; baseline (speedup 1.0000x reference)
import jax
import jax.numpy as jnp
from jax.experimental import pallas as pl


def kernel(bi_loc_pred, bi_conf_pred, multi_loc_pred, multi_conf_pred, priors, targets):
    raise NotImplementedError("write your pallas kernel here")


# speedup vs baseline: 16.9719x; 16.9719x over previous
import jax
import jax.numpy as jnp
from jax.experimental import pallas as pl
from jax.experimental.pallas import tpu as pltpu

NUM_CLASSES = 21
OVERLAP_THRESH = 0.5
NEG_POS_RATIO = 3
ARM_VARIANCE = (0.1, 0.2)
VARIANCE = (0.1, 0.2)
POS_PRIOR_THRESHOLD = 0.01
T = 8


def _sortable_key(v):
    b = jax.lax.bitcast_convert_type(v, jnp.int32)
    return jnp.where(b >= 0, b, b ^ jnp.int32(0x7FFFFFFF))


def _odm_kernel(tgt_ref, bi_loc_ref, bi_conf_ref, multi_loc_ref,
                multi_conf_ref, priors_ref, ll_ref, lc_ref, np_ref):
    P = bi_loc_ref.shape[2]
    f32 = jnp.float32

    @pl.when(pl.program_id(0) == 0)
    def _init():
        ll_ref[0, 0] = f32(0.0)
        lc_ref[0, 0] = f32(0.0)
        np_ref[0, 0] = f32(0.0)

    al = bi_loc_ref[0]
    ml = multi_loc_ref[0]
    pr = priors_ref[...]
    bc = bi_conf_ref[0]
    mc = multi_conf_ref[0]

    cx = pr[0:1] + al[0:1] * (ARM_VARIANCE[0] * pr[2:3])
    cy = pr[1:2] + al[1:2] * (ARM_VARIANCE[0] * pr[3:4])
    w = pr[2:3] * jnp.exp(al[2:3] * ARM_VARIANCE[1])
    h = pr[3:4] * jnp.exp(al[3:4] * ARM_VARIANCE[1])
    x0 = cx - w / 2.0
    y0 = cy - h / 2.0
    x1 = cx + w / 2.0
    y1 = cy + h / 2.0
    area_p = (x1 - x0) * (y1 - y0)

    tx0 = [tgt_ref[0, t, 0] for t in range(T)]
    ty0 = [tgt_ref[0, t, 1] for t in range(T)]
    tx1 = [tgt_ref[0, t, 2] for t in range(T)]
    ty1 = [tgt_ref[0, t, 3] for t in range(T)]
    tlb = [tgt_ref[0, t, 4] for t in range(T)]

    ious = []
    for t in range(T):
        iw = jnp.maximum(jnp.minimum(tx1[t], x1) - jnp.maximum(tx0[t], x0), 0.0)
        ih = jnp.maximum(jnp.minimum(ty1[t], y1) - jnp.maximum(ty0[t], y0), 0.0)
        inter = iw * ih
        area_t = (tx1[t] - tx0[t]) * (ty1[t] - ty0[t])
        ious.append(inter / (area_t + area_p - inter))

    bt_over = ious[0]
    bt_idx = jnp.zeros((1, P), jnp.int32)
    for t in range(1, T):
        upd = ious[t] > bt_over
        bt_idx = jnp.where(upd, jnp.int32(t), bt_idx)
        bt_over = jnp.where(upd, ious[t], bt_over)

    lane = jax.lax.broadcasted_iota(jnp.int32, (1, P), 1)
    for t in range(T):
        mval = jnp.max(ious[t])
        bpi = jnp.min(jnp.where(ious[t] == mval, lane, jnp.int32(P)))
        hit = lane == bpi
        bt_over = jnp.where(hit, f32(2.0), bt_over)
        bt_idx = jnp.where(hit, jnp.int32(t), bt_idx)

    m0 = jnp.zeros((1, P), f32)
    m1 = jnp.zeros((1, P), f32)
    m2 = jnp.zeros((1, P), f32)
    m3 = jnp.zeros((1, P), f32)
    lbl = jnp.zeros((1, P), f32)
    for t in range(T):
        sel_t = bt_idx == t
        m0 = jnp.where(sel_t, tx0[t], m0)
        m1 = jnp.where(sel_t, ty0[t], m1)
        m2 = jnp.where(sel_t, tx1[t], m2)
        m3 = jnp.where(sel_t, ty1[t], m3)
        lbl = jnp.where(sel_t, tlb[t], lbl)

    conf_t = lbl.astype(jnp.int32) + 1
    conf_t = jnp.where(bt_over < OVERLAP_THRESH, 0, conf_t)
    pos = conf_t > 0
    posf = pos.astype(f32)
    num_pos = jnp.sum(conf_t > 0, dtype=jnp.int32)

    g0 = ((m0 + m2) / 2.0 - cx) / (VARIANCE[0] * w)
    g1 = ((m1 + m3) / 2.0 - cy) / (VARIANCE[0] * h)
    g2 = jnp.log((m2 - m0) / w) / VARIANCE[1]
    g3 = jnp.log((m3 - m1) / h) / VARIANCE[1]

    loss_l = f32(0.0)
    for d, g in ((ml[0:1], g0), (ml[1:2], g1), (ml[2:3], g2), (ml[3:4], g3)):
        df = d - g
        ad = jnp.abs(df)
        sl1 = jnp.where(ad < 1.0, 0.5 * df * df, ad - 0.5)
        loss_l = loss_l + jnp.sum(sl1 * posf)

    cmax = jnp.max(mc, axis=0, keepdims=True)
    ez = jnp.exp(mc - cmax)
    lse = jnp.log(jnp.sum(ez, axis=0, keepdims=True)) + cmax
    cls = jax.lax.broadcasted_iota(jnp.int32, (NUM_CLASSES, P), 0)
    onehot = (cls == conf_t).astype(f32)
    gathered = jnp.sum(mc * onehot, axis=0, keepdims=True)
    ce = lse - gathered

    amax = jnp.maximum(bc[0:1], bc[1:2])
    e0 = jnp.exp(bc[0:1] - amax)
    e1 = jnp.exp(bc[1:2] - amax)
    score1 = e1 / (e0 + e1)

    v = jnp.where(pos, f32(0.0), ce)
    v = jnp.where(jnp.logical_and(conf_t <= 0, score1 < POS_PRIOR_THRESHOLD),
                  f32(0.0), v)

    k = jnp.minimum(NEG_POS_RATIO * num_pos, P - 1)
    skey = _sortable_key(v)

    def bit_body(i, tbits):
        cand_bits = tbits | (jnp.int32(1) << (31 - i))
        cand = cand_bits ^ jnp.int32(-2147483648)
        cnt = jnp.sum((skey >= cand).astype(jnp.int32))
        return jnp.where(cnt >= k, cand_bits, tbits)

    tbits = jax.lax.fori_loop(0, 32, bit_body, jnp.int32(0))
    tkey = tbits ^ jnp.int32(-2147483648)
    c_gt = jnp.sum((skey > tkey).astype(jnp.int32))
    eq = skey == tkey
    eqi = eq.astype(jnp.int32)
    r = k - c_gt

    def idx_body(i, x):
        cand = x | (jnp.int32(1) << (15 - i))
        cnt = jnp.sum(jnp.where(lane < cand, eqi, 0))
        return jnp.where(cnt < r, cand, x)

    xcut = jax.lax.fori_loop(0, 16, idx_body, jnp.int32(0))
    neg = jnp.logical_or(skey > tkey,
                         jnp.logical_and(eq, lane <= xcut))
    neg = jnp.logical_and(neg, k > 0)

    selm = jnp.logical_or(pos, neg).astype(f32)
    loss_c = jnp.sum(ce * selm)

    ll_ref[0, 0] += loss_l
    lc_ref[0, 0] += loss_c
    np_ref[0, 0] += num_pos.astype(f32)


@jax.jit
def _odm_loss_impl(bi_loc_pred, bi_conf_pred, multi_loc_pred, multi_conf_pred,
                   priors, targets):
    B, P, _ = bi_loc_pred.shape
    bi_loc_t = jnp.transpose(jax.lax.stop_gradient(bi_loc_pred), (0, 2, 1))
    bi_conf_t = jnp.transpose(jax.lax.stop_gradient(bi_conf_pred), (0, 2, 1))
    multi_loc_t = jnp.transpose(multi_loc_pred, (0, 2, 1))
    multi_conf_t = jnp.transpose(multi_conf_pred, (0, 2, 1))
    priors_t = jnp.transpose(jax.lax.stop_gradient(priors), (1, 0))
    targets_d = jax.lax.stop_gradient(targets)

    ll, lc, npos = pl.pallas_call(
        _odm_kernel,
        grid=(B,),
        in_specs=[
            pl.BlockSpec((1, T, 5), lambda b: (b, 0, 0),
                         memory_space=pltpu.SMEM),
            pl.BlockSpec((1, 4, P), lambda b: (b, 0, 0)),
            pl.BlockSpec((1, 2, P), lambda b: (b, 0, 0)),
            pl.BlockSpec((1, 4, P), lambda b: (b, 0, 0)),
            pl.BlockSpec((1, NUM_CLASSES, P), lambda b: (b, 0, 0)),
            pl.BlockSpec((4, P), lambda b: (0, 0)),
        ],
        out_specs=[
            pl.BlockSpec((1, 1), lambda b: (0, 0), memory_space=pltpu.SMEM),
            pl.BlockSpec((1, 1), lambda b: (0, 0), memory_space=pltpu.SMEM),
            pl.BlockSpec((1, 1), lambda b: (0, 0), memory_space=pltpu.SMEM),
        ],
        out_shape=[
            jax.ShapeDtypeStruct((1, 1), jnp.float32),
            jax.ShapeDtypeStruct((1, 1), jnp.float32),
            jax.ShapeDtypeStruct((1, 1), jnp.float32),
        ],
    )(targets_d, bi_loc_t, bi_conf_t, multi_loc_t, multi_conf_t, priors_t)

    total = npos[0, 0]
    return ll[0, 0] / total, lc[0, 0] / total


def kernel(bi_loc_pred, bi_conf_pred, multi_loc_pred, multi_conf_pred,
           priors, targets):
    return _odm_loss_impl(bi_loc_pred, bi_conf_pred, multi_loc_pred,
                          multi_conf_pred, priors, targets)
